# Initial kernel scaffold; baseline (speedup 1.0000x reference)
import jax
import jax.numpy as jnp
from jax.experimental import pallas as pl


def kernel(x, edge_index, Wl1, bl1, Wr1, Wl2, bl2, Wr2):
    raise NotImplementedError("write your pallas kernel here")


# speedup vs baseline: 2.9975x; 2.9975x over previous
import jax
import jax.numpy as jnp
from jax import lax
from jax.experimental import pallas as pl
from jax.experimental.pallas import tpu as pltpu
from jax.experimental.pallas import tpu_sc as plsc

_N = 10000
_E = 160000
_NC = 2
_NS = 16
_NP = 10240
_EP = 163840
_NB = 80
_B = 128
_DUMMY = 10100
_RPT = _NP // _NS
_D = 128


def _agg_body(table, srcf, dsti, zrow, out, acc, idx_sf, idx_df, idx_s,
              idx_d, buf, sem):
    c = lax.axis_index("c")
    s = lax.axis_index("s")

    base = s * _RPT
    pltpu.sync_copy(zrow, acc.at[pl.ds(base, _RPT)])
    plsc.subcore_barrier()

    sbase = (c * _NS + s) * _NB
    dbase = s * _NB

    def step(t, cc):
        pltpu.sync_copy(srcf.at[pl.ds(sbase + t, 1)], idx_sf)
        pltpu.sync_copy(dsti.at[pl.ds(dbase + t, 1)], idx_df)
        for k in range(_B // 16):
            sl = pl.ds(k * 16, 16)
            idx_s[0, sl] = idx_sf[0, sl].astype(jnp.int32)
            idx_d[0, sl] = idx_df[0, sl].astype(jnp.int32)
        pltpu.async_copy(table.at[idx_s.at[0]], buf, sem).wait()
        pltpu.sync_copy(buf, acc.at[idx_d.at[0]], add=True)
        return cc

    lax.fori_loop(0, _NB, step, 0)
    plsc.subcore_barrier()

    pltpu.sync_copy(acc.at[pl.ds(base, _RPT)], out.at[c, pl.ds(base, _RPT)])


_sc_agg = pl.kernel(
    _agg_body,
    out_type=jax.ShapeDtypeStruct((_NC, _NP, _D), jnp.float32),
    mesh=plsc.VectorSubcoreMesh(core_axis_name="c", subcore_axis_name="s"),
    scratch_types=[
        pltpu.VMEM_SHARED((_NP, _D), jnp.float32),
        pltpu.VMEM((1, _B), jnp.float32),
        pltpu.VMEM((1, _B), jnp.float32),
        pltpu.VMEM((1, _B), jnp.int32),
        pltpu.VMEM((1, _B), jnp.int32),
        pltpu.VMEM((_B, _D), jnp.float32),
        pltpu.SemaphoreType.DMA,
    ],
)


def _deg_body(dsti, zrow, onesr, deg_out, acc, idx_df, idx_d, ones_b):
    c = lax.axis_index("c")
    s = lax.axis_index("s")

    base = s * _RPT
    pltpu.sync_copy(zrow, acc.at[pl.ds(base, _RPT)])
    pltpu.sync_copy(onesr, ones_b)
    plsc.subcore_barrier()

    nb2 = _NB // _NC
    dbase = (c * _NS + s) * nb2

    def step(t, cc):
        pltpu.sync_copy(dsti.at[pl.ds(dbase + t, 1)], idx_df)
        for k in range(_B // 16):
            sl = pl.ds(k * 16, 16)
            idx_d[0, sl] = idx_df[0, sl].astype(jnp.int32)
        pltpu.sync_copy(ones_b, acc.at[idx_d.at[0]], add=True)
        return cc

    lax.fori_loop(0, nb2, step, 0)
    plsc.subcore_barrier()

    pltpu.sync_copy(acc.at[pl.ds(base, _RPT)],
                    deg_out.at[c, pl.ds(base, _RPT)])


_sc_deg = pl.kernel(
    _deg_body,
    out_type=jax.ShapeDtypeStruct((_NC, _NP, _D), jnp.float32),
    mesh=plsc.VectorSubcoreMesh(core_axis_name="c", subcore_axis_name="s"),
    scratch_types=[
        pltpu.VMEM_SHARED((_NP, _D), jnp.float32),
        pltpu.VMEM((1, _B), jnp.float32),
        pltpu.VMEM((1, _B), jnp.int32),
        pltpu.VMEM((_B, _D), jnp.float32),
    ],
)

_R = 512
_GB = _NP // _R


def _tc_mid(agg1, degs, xpad, w1l2, bl1, wr1t, wl2t, wr2t):
    def body(agg_ref, deg_ref, x_ref, wl_ref, bl_ref, wr_ref, w2l_ref,
             w2r_ref, p2_ref, r_ref):
        deg = deg_ref[0][:, 0:1] + deg_ref[1][:, 0:1]
        inv = 1.0 / jnp.maximum(deg, 1.0)
        a0 = agg_ref[0] * inv
        a1 = agg_ref[1] * inv
        h = (jnp.dot(a0, wl_ref[0], preferred_element_type=jnp.float32)
             + jnp.dot(a1, wl_ref[1], preferred_element_type=jnp.float32)
             + jnp.dot(x_ref[...], wr_ref[...],
                       preferred_element_type=jnp.float32)
             + bl_ref[...])
        h = jnp.maximum(h, 0.0)
        p = jnp.dot(h, w2l_ref[...], preferred_element_type=jnp.float32)
        r_ref[...] = jnp.dot(h, w2r_ref[...],
                             preferred_element_type=jnp.float32)
        p2_ref[0] = p[:, :128]
        p2_ref[1] = p[:, 128:]

    return pl.pallas_call(
        body,
        grid=(_GB,),
        in_specs=[
            pl.BlockSpec((_NC, _R, 128), lambda i: (0, i, 0)),
            pl.BlockSpec((_NC, _R, 128), lambda i: (0, i, 0)),
            pl.BlockSpec((_R, 256), lambda i: (i, 0)),
            pl.BlockSpec((_NC, 128, 512), lambda i: (0, 0, 0)),
            pl.BlockSpec((1, 512), lambda i: (0, 0)),
            pl.BlockSpec((256, 512), lambda i: (0, 0)),
            pl.BlockSpec((512, 256), lambda i: (0, 0)),
            pl.BlockSpec((512, 256), lambda i: (0, 0)),
        ],
        out_specs=[
            pl.BlockSpec((_NC, _R, 128), lambda i: (0, i, 0)),
            pl.BlockSpec((_R, 256), lambda i: (i, 0)),
        ],
        out_shape=[
            jax.ShapeDtypeStruct((_NC, _NP, 128), jnp.float32),
            jax.ShapeDtypeStruct((_NP, 256), jnp.float32),
        ],
    )(agg1, degs, xpad, w1l2, bl1, wr1t, wl2t, wr2t)


def _tc_out(agg2, degs, r, bl2):
    def body(agg_ref, deg_ref, r_ref, bl_ref, out_ref):
        deg = deg_ref[0][:, 0:1] + deg_ref[1][:, 0:1]
        inv = 1.0 / jnp.maximum(deg, 1.0)
        out_ref[:, :128] = agg_ref[0] * inv + bl_ref[:, :128] + r_ref[:, :128]
        out_ref[:, 128:] = agg_ref[1] * inv + bl_ref[:, 128:] + r_ref[:, 128:]

    return pl.pallas_call(
        body,
        grid=(_GB,),
        in_specs=[
            pl.BlockSpec((_NC, _R, 128), lambda i: (0, i, 0)),
            pl.BlockSpec((_NC, _R, 128), lambda i: (0, i, 0)),
            pl.BlockSpec((_R, 256), lambda i: (i, 0)),
            pl.BlockSpec((1, 256), lambda i: (0, 0)),
        ],
        out_specs=pl.BlockSpec((_R, 256), lambda i: (i, 0)),
        out_shape=jax.ShapeDtypeStruct((_NP, 256), jnp.float32),
    )(agg2, degs, r, bl2)


def kernel(x, edge_index, Wl1, bl1, Wr1, Wl2, bl2, Wr2):
    src = edge_index[0]
    dst = edge_index[1]
    srcp = jnp.concatenate([src, jnp.zeros((_EP - _E,), jnp.int32)])
    dstp = jnp.concatenate([dst, jnp.full((_EP - _E,), _DUMMY, jnp.int32)])
    srcf = jnp.concatenate([srcp, srcp + _NP]).reshape(-1, _B).astype(jnp.float32)
    dsti = dstp.reshape(-1, _B).astype(jnp.float32)

    xpad = jnp.pad(x, ((0, _NP - _N), (0, 0)))
    xh = xpad.reshape(_NP, 2, 128).transpose(1, 0, 2).reshape(2 * _NP, 128)

    zrow = jnp.zeros((_RPT, _D), jnp.float32)
    onesr = jnp.ones((_B, _D), jnp.float32)

    degs = _sc_deg(dsti, zrow, onesr)
    agg1 = _sc_agg(xh, srcf, dsti, zrow)

    w1l2 = jnp.stack([Wl1.T[:128], Wl1.T[128:]])
    p2, r = _tc_mid(agg1, degs, xpad, w1l2, bl1.reshape(1, 512),
                    Wr1.T, Wl2.T, Wr2.T)

    ph = p2.reshape(2 * _NP, 128)
    agg2 = _sc_agg(ph, srcf, dsti, zrow)
    out = _tc_out(agg2, degs, r, bl2.reshape(1, 256))
    return out[:_N]


# speedup vs baseline: 3.8000x; 1.2678x over previous
import jax
import jax.numpy as jnp
from jax import lax
from jax.experimental import pallas as pl
from jax.experimental.pallas import tpu as pltpu
from jax.experimental.pallas import tpu_sc as plsc

_N = 10000
_E = 160000
_NC = 2
_NS = 16
_NP = 10240
_EP = 163840
_NB = 80
_B = 128
_DUMMY = 10100
_RPT = _NP // _NS
_D = 128


_G = 8


def _agg_body(table, srcf, dsti, zrow, out, acc, idx_sf, idx_df, idx_s,
              idx_d, buf, gsem0, gsem1):
    c = lax.axis_index("c")
    s = lax.axis_index("s")

    base = s * _RPT
    pltpu.sync_copy(zrow, acc.at[pl.ds(base, _RPT)])
    plsc.subcore_barrier()

    sbase = (c * _NS + s) * _NB
    dbase = s * _NB
    gsems = (gsem0, gsem1)

    def chunk(g, cc):
        pltpu.sync_copy(srcf.at[pl.ds(sbase + g * _G, _G)], idx_sf)
        pltpu.sync_copy(dsti.at[pl.ds(dbase + g * _G, _G)], idx_df)
        for j in range(_G):
            for k in range(_B // 16):
                sl = pl.ds(k * 16, 16)
                idx_s[j, sl] = idx_sf[j, sl].astype(jnp.int32)
                idx_d[j, sl] = idx_df[j, sl].astype(jnp.int32)
        descs = {0: pltpu.async_copy(table.at[idx_s.at[0]], buf.at[0],
                                     gsems[0])}
        for b in range(_G):
            if b + 1 < _G:
                descs[b + 1] = pltpu.async_copy(
                    table.at[idx_s.at[b + 1]], buf.at[(b + 1) % 2],
                    gsems[(b + 1) % 2])
            descs[b].wait()
            pltpu.sync_copy(buf.at[b % 2], acc.at[idx_d.at[b]], add=True)
        return cc

    lax.fori_loop(0, _NB // _G, chunk, 0)
    plsc.subcore_barrier()

    pltpu.sync_copy(acc.at[pl.ds(base, _RPT)], out.at[c, pl.ds(base, _RPT)])


_sc_agg = pl.kernel(
    _agg_body,
    out_type=jax.ShapeDtypeStruct((_NC, _NP, _D), jnp.float32),
    mesh=plsc.VectorSubcoreMesh(core_axis_name="c", subcore_axis_name="s"),
    scratch_types=[
        pltpu.VMEM_SHARED((_NP, _D), jnp.float32),
        pltpu.VMEM((_G, _B), jnp.float32),
        pltpu.VMEM((_G, _B), jnp.float32),
        pltpu.VMEM((_G, _B), jnp.int32),
        pltpu.VMEM((_G, _B), jnp.int32),
        pltpu.VMEM((2, _B, _D), jnp.float32),
        pltpu.SemaphoreType.DMA,
        pltpu.SemaphoreType.DMA,
    ],
)


def _deg_body(dsti, zrow, onesr, deg_out, acc, idx_df, idx_d, ones_b):
    c = lax.axis_index("c")
    s = lax.axis_index("s")

    base = s * _RPT
    pltpu.sync_copy(zrow, acc.at[pl.ds(base, _RPT)])
    pltpu.sync_copy(onesr, ones_b)
    plsc.subcore_barrier()

    nb2 = _NB // _NC
    dbase = (c * _NS + s) * nb2

    def step(t, cc):
        pltpu.sync_copy(dsti.at[pl.ds(dbase + t, 1)], idx_df)
        for k in range(_B // 16):
            sl = pl.ds(k * 16, 16)
            idx_d[0, sl] = idx_df[0, sl].astype(jnp.int32)
        pltpu.sync_copy(ones_b, acc.at[idx_d.at[0]], add=True)
        return cc

    lax.fori_loop(0, nb2, step, 0)
    plsc.subcore_barrier()

    pltpu.sync_copy(acc.at[pl.ds(base, _RPT)],
                    deg_out.at[c, pl.ds(base, _RPT)])


_sc_deg = pl.kernel(
    _deg_body,
    out_type=jax.ShapeDtypeStruct((_NC, _NP, _D), jnp.float32),
    mesh=plsc.VectorSubcoreMesh(core_axis_name="c", subcore_axis_name="s"),
    scratch_types=[
        pltpu.VMEM_SHARED((_NP, _D), jnp.float32),
        pltpu.VMEM((1, _B), jnp.float32),
        pltpu.VMEM((1, _B), jnp.int32),
        pltpu.VMEM((_B, _D), jnp.float32),
    ],
)

_R = 512
_GB = _NP // _R


def _tc_mid(agg1, degs, xpad, w1l2, bl1, wr1t, wl2t, wr2t):
    def body(agg_ref, deg_ref, x_ref, wl_ref, bl_ref, wr_ref, w2l_ref,
             w2r_ref, p2_ref, r_ref):
        deg = deg_ref[0][:, 0:1] + deg_ref[1][:, 0:1]
        inv = 1.0 / jnp.maximum(deg, 1.0)
        a0 = agg_ref[0] * inv
        a1 = agg_ref[1] * inv
        h = (jnp.dot(a0, wl_ref[0], preferred_element_type=jnp.float32)
             + jnp.dot(a1, wl_ref[1], preferred_element_type=jnp.float32)
             + jnp.dot(x_ref[...], wr_ref[...],
                       preferred_element_type=jnp.float32)
             + bl_ref[...])
        h = jnp.maximum(h, 0.0)
        p = jnp.dot(h, w2l_ref[...], preferred_element_type=jnp.float32)
        r_ref[...] = jnp.dot(h, w2r_ref[...],
                             preferred_element_type=jnp.float32)
        p2_ref[0] = p[:, :128]
        p2_ref[1] = p[:, 128:]

    return pl.pallas_call(
        body,
        grid=(_GB,),
        in_specs=[
            pl.BlockSpec((_NC, _R, 128), lambda i: (0, i, 0)),
            pl.BlockSpec((_NC, _R, 128), lambda i: (0, i, 0)),
            pl.BlockSpec((_R, 256), lambda i: (i, 0)),
            pl.BlockSpec((_NC, 128, 512), lambda i: (0, 0, 0)),
            pl.BlockSpec((1, 512), lambda i: (0, 0)),
            pl.BlockSpec((256, 512), lambda i: (0, 0)),
            pl.BlockSpec((512, 256), lambda i: (0, 0)),
            pl.BlockSpec((512, 256), lambda i: (0, 0)),
        ],
        out_specs=[
            pl.BlockSpec((_NC, _R, 128), lambda i: (0, i, 0)),
            pl.BlockSpec((_R, 256), lambda i: (i, 0)),
        ],
        out_shape=[
            jax.ShapeDtypeStruct((_NC, _NP, 128), jnp.float32),
            jax.ShapeDtypeStruct((_NP, 256), jnp.float32),
        ],
    )(agg1, degs, xpad, w1l2, bl1, wr1t, wl2t, wr2t)


def _tc_out(agg2, degs, r, bl2):
    def body(agg_ref, deg_ref, r_ref, bl_ref, out_ref):
        deg = deg_ref[0][:, 0:1] + deg_ref[1][:, 0:1]
        inv = 1.0 / jnp.maximum(deg, 1.0)
        out_ref[:, :128] = agg_ref[0] * inv + bl_ref[:, :128] + r_ref[:, :128]
        out_ref[:, 128:] = agg_ref[1] * inv + bl_ref[:, 128:] + r_ref[:, 128:]

    return pl.pallas_call(
        body,
        grid=(_GB,),
        in_specs=[
            pl.BlockSpec((_NC, _R, 128), lambda i: (0, i, 0)),
            pl.BlockSpec((_NC, _R, 128), lambda i: (0, i, 0)),
            pl.BlockSpec((_R, 256), lambda i: (i, 0)),
            pl.BlockSpec((1, 256), lambda i: (0, 0)),
        ],
        out_specs=pl.BlockSpec((_R, 256), lambda i: (i, 0)),
        out_shape=jax.ShapeDtypeStruct((_NP, 256), jnp.float32),
    )(agg2, degs, r, bl2)


def kernel(x, edge_index, Wl1, bl1, Wr1, Wl2, bl2, Wr2):
    src = edge_index[0]
    dst = edge_index[1]
    srcp = jnp.concatenate([src, jnp.zeros((_EP - _E,), jnp.int32)])
    dstp = jnp.concatenate([dst, jnp.full((_EP - _E,), _DUMMY, jnp.int32)])
    srcf = jnp.concatenate([srcp, srcp + _NP]).reshape(-1, _B).astype(jnp.float32)
    dsti = dstp.reshape(-1, _B).astype(jnp.float32)

    xpad = jnp.pad(x, ((0, _NP - _N), (0, 0)))
    xh = xpad.reshape(_NP, 2, 128).transpose(1, 0, 2).reshape(2 * _NP, 128)

    zrow = jnp.zeros((_RPT, _D), jnp.float32)
    onesr = jnp.ones((_B, _D), jnp.float32)

    degs = _sc_deg(dsti, zrow, onesr)
    agg1 = _sc_agg(xh, srcf, dsti, zrow)

    w1l2 = jnp.stack([Wl1.T[:128], Wl1.T[128:]])
    p2, r = _tc_mid(agg1, degs, xpad, w1l2, bl1.reshape(1, 512),
                    Wr1.T, Wl2.T, Wr2.T)

    ph = p2.reshape(2 * _NP, 128)
    agg2 = _sc_agg(ph, srcf, dsti, zrow)
    out = _tc_out(agg2, degs, r, bl2.reshape(1, 256))
    return out[:_N]
